# baseline (speedup 1.0000x reference)
import functools

import numpy as np
import jax
import jax.numpy as jnp
from jax import lax
from jax.experimental import pallas as pl
from jax.experimental.pallas import tpu as pltpu
from jax.experimental.pallas import tpu_sc as plsc


_RES = 64
_BM = 2048
_NW = 32
_NT = 16
_CH = 512
_TSZ = _RES ** 3 + 1024
_SENT = _RES ** 3
_JUNK = _RES ** 3 + 2


def _child_parent_maps():
    A = np.zeros((2, 3, 3), dtype=np.float32)
    A[0, 0, 0] = 1.0
    A[0, 1, 1] = A[0, 1, 2] = 1.0
    A[1, 1, 0] = A[1, 1, 1] = 1.0
    A[1, 2, 2] = 1.0
    M = np.zeros((8, 27, 27), dtype=np.float32)
    for ox in range(2):
        for oy in range(2):
            for oz in range(2):
                o = 4 * ox + 2 * oy + oz
                for ex in range(3):
                    for ey in range(3):
                        for ez in range(3):
                            e = 9 * ex + 3 * ey + ez
                            for dx in range(3):
                                for dy in range(3):
                                    for dz in range(3):
                                        d = 9 * dx + 3 * dy + dz
                                        M[o, e, d] = (A[ox, ex, dx]
                                                      * A[oy, ey, dy]
                                                      * A[oz, ez, dz])
    return jnp.asarray(M)


def _as_i32_rows(x_bf16):
    r, c = x_bf16.shape
    return lax.bitcast_convert_type(
        x_bf16.reshape(r, c // 2, 2), jnp.int32)


def _sc_conv1_gather(nkx, keys_pad, ids, feats_i32, ztab, npad):
    rtot = npad * 27
    nch = rtot // _CH
    steps = -(-nch // _NW)
    rf = feats_i32.shape[0]
    ts, fs, vs = _TSZ // _CH, rf // _CH, npad // _NT
    mesh = plsc.VectorSubcoreMesh(core_axis_name="c", subcore_axis_name="s")

    @functools.partial(
        pl.kernel, mesh=mesh,
        out_type=(jax.ShapeDtypeStruct((rtot,), jnp.int32),
                  jax.ShapeDtypeStruct((rtot, 16), jnp.int32)),
        compiler_params=pltpu.CompilerParams(use_tc_tiling_on_sc=False),
        scratch_types=[
            pltpu.VMEM_SHARED((_TSZ,), jnp.int32),
            pltpu.VMEM_SHARED((rf, 16), jnp.int32),
            pltpu.VMEM((vs // 4,), jnp.int32),
            pltpu.VMEM((vs // 4,), jnp.int32),
            pltpu.VMEM((_CH,), jnp.int32),
            pltpu.VMEM((_CH,), jnp.int32),
            pltpu.VMEM((_CH, 16), jnp.int32),
            pltpu.SemaphoreType.DMA,
        ],
    )
    def k(nkx_hbm, keys_hbm, ids_hbm, feats_hbm, ztab_hbm, g_hbm, gout_hbm,
          tab_sh, feats_sh, keys_v, ids_v, nk_v, g_v, rows_v, sem):
        sid = lax.axis_index("s")
        wid = sid * 2 + lax.axis_index("c")

        def zstage(t, carry):
            c = sid + t * _NT

            @pl.when(c < ts)
            def _():
                pltpu.sync_copy(ztab_hbm.at[pl.ds(c * _CH, _CH)], g_v)
                pltpu.sync_copy(g_v, tab_sh.at[pl.ds(c * _CH, _CH)])

            return carry

        def fstage(t, carry):
            c = sid + t * _NT

            @pl.when(c < fs)
            def _():
                pltpu.sync_copy(feats_hbm.at[pl.ds(c * _CH, _CH)], rows_v)
                pltpu.sync_copy(rows_v, feats_sh.at[pl.ds(c * _CH, _CH)])

            return carry

        lax.fori_loop(0, -(-ts // _NT), zstage, 0)
        lax.fori_loop(0, -(-fs // _NT), fstage, 0)
        plsc.subcore_barrier()

        def scat(p, carry):
            off = sid * vs + p * (vs // 4)
            pltpu.sync_copy(keys_hbm.at[pl.ds(off, vs // 4)], keys_v)
            pltpu.sync_copy(ids_hbm.at[pl.ds(off, vs // 4)], ids_v)
            pltpu.async_copy(ids_v, tab_sh.at[keys_v], sem).wait()
            return carry

        lax.fori_loop(0, 4, scat, 0)
        plsc.subcore_barrier()

        def step(t, carry):
            c = wid + t * _NW

            @pl.when(c < nch)
            def _():
                pltpu.sync_copy(nkx_hbm.at[pl.ds(c * _CH, _CH)], nk_v)
                pltpu.async_copy(tab_sh.at[nk_v], g_v, sem).wait()
                pltpu.async_copy(feats_sh.at[g_v], rows_v, sem).wait()
                pltpu.sync_copy(g_v, g_hbm.at[pl.ds(c * _CH, _CH)])
                pltpu.sync_copy(rows_v, gout_hbm.at[pl.ds(c * _CH, _CH)])

            return carry

        lax.fori_loop(0, steps, step, 0)

    return k(nkx, keys_pad, ids, feats_i32, ztab)


def _sc_conv2_gather(g, h1_i32, npad):
    rtot = npad * 27
    nch = rtot // _CH
    steps = -(-nch // _NW)
    rf = h1_i32.shape[0]
    fs = rf // _CH
    mesh = plsc.VectorSubcoreMesh(core_axis_name="c", subcore_axis_name="s")

    @functools.partial(
        pl.kernel, mesh=mesh,
        out_type=jax.ShapeDtypeStruct((rtot, 16), jnp.int32),
        compiler_params=pltpu.CompilerParams(use_tc_tiling_on_sc=False),
        scratch_types=[
            pltpu.VMEM_SHARED((rf, 16), jnp.int32),
            pltpu.VMEM((_CH,), jnp.int32),
            pltpu.VMEM((_CH, 16), jnp.int32),
            pltpu.SemaphoreType.DMA,
        ],
    )
    def k(g_hbm, h_hbm, gout_hbm, h_sh, g_v, rows_v, sem):
        sid = lax.axis_index("s")
        wid = sid * 2 + lax.axis_index("c")

        def fstage(t, carry):
            c = sid + t * _NT

            @pl.when(c < fs)
            def _():
                pltpu.sync_copy(h_hbm.at[pl.ds(c * _CH, _CH)], rows_v)
                pltpu.sync_copy(rows_v, h_sh.at[pl.ds(c * _CH, _CH)])

            return carry

        lax.fori_loop(0, -(-fs // _NT), fstage, 0)
        plsc.subcore_barrier()

        def step(t, carry):
            c = wid + t * _NW

            @pl.when(c < nch)
            def _():
                pltpu.sync_copy(g_hbm.at[pl.ds(c * _CH, _CH)], g_v)
                pltpu.async_copy(h_sh.at[g_v], rows_v, sem).wait()
                pltpu.sync_copy(rows_v, gout_hbm.at[pl.ds(c * _CH, _CH)])

            return carry

        lax.fori_loop(0, steps, step, 0)

    return k(g, h1_i32)


def _matmul_silu_kernel(x_ref, w_ref, b_ref, o_ref):
    z = jnp.dot(x_ref[...], w_ref[...],
                preferred_element_type=jnp.float32) + b_ref[...]
    o_ref[...] = z * (1.0 / (1.0 + jnp.exp(-z)))


def _matmul_silu(x, w, b):
    m, k = x.shape
    co = w.shape[1]
    return pl.pallas_call(
        _matmul_silu_kernel,
        grid=(m // _BM,),
        in_specs=[
            pl.BlockSpec((_BM, k), lambda i: (i, 0)),
            pl.BlockSpec((k, co), lambda i: (0, 0)),
            pl.BlockSpec((1, co), lambda i: (0, 0)),
        ],
        out_specs=pl.BlockSpec((_BM, co), lambda i: (i, 0)),
        out_shape=jax.ShapeDtypeStruct((m, co), jnp.float32),
    )(x, w, b.reshape(1, co))


def kernel(feats, W1, b1, W2, b2, coords):
    n, cin = feats.shape
    cout = W2.shape[2]
    res = _RES
    npad = ((n + _BM - 1) // _BM) * _BM
    rf = npad + 512

    x, y, z = coords[:, 1], coords[:, 2], coords[:, 3]
    keys = (x * res + y) * res + z
    offs = np.array([[dx, dy, dz]
                     for dx in (-1, 0, 1)
                     for dy in (-1, 0, 1)
                     for dz in (-1, 0, 1)], dtype=np.int32)
    delta = jnp.asarray(offs[:, 0] * res * res + offs[:, 1] * res + offs[:, 2])
    q = coords[:, None, 1:4] + jnp.asarray(offs)[None, :, :]
    valid = jnp.all((q >= 0) & (q < res), axis=-1)
    nkx = jnp.where(valid, keys[:, None] + delta[None, :], _SENT)
    nkx = jnp.pad(nkx, ((0, npad - n), (0, 0)),
                  constant_values=_SENT).reshape(-1)

    keys_pad = jnp.pad(keys, (0, npad - n), constant_values=_JUNK)
    ids = jnp.arange(1, npad + 1, dtype=jnp.int32)
    ztab = jnp.zeros((_TSZ,), jnp.int32)

    fimg = jnp.concatenate(
        [jnp.zeros((1, cin), jnp.bfloat16),
         feats.astype(jnp.bfloat16),
         jnp.zeros((rf - 1 - n, cin), jnp.bfloat16)], 0)
    g, G1 = _sc_conv1_gather(nkx, keys_pad, ids, _as_i32_rows(fimg),
                             ztab, npad)
    G1 = lax.bitcast_convert_type(G1, jnp.bfloat16).reshape(npad, 27 * cin)
    W1r = W1.reshape(27 * cin, W1.shape[2]).astype(jnp.bfloat16)
    h1 = _matmul_silu(G1, W1r, b1)

    M = _child_parent_maps()
    W2eff = jnp.einsum('oed,dij->eioj', M, W2).reshape(27 * cin, 8 * cout)
    b2t = jnp.tile(b2, 8)
    himg = jnp.concatenate(
        [jnp.zeros((1, cin), jnp.bfloat16),
         h1.astype(jnp.bfloat16),
         jnp.zeros((rf - 1 - npad, cin), jnp.bfloat16)], 0)
    G2 = _sc_conv2_gather(g, _as_i32_rows(himg), npad)
    G2 = lax.bitcast_convert_type(G2, jnp.bfloat16).reshape(npad, 27 * cin)
    out = _matmul_silu(G2, W2eff.astype(jnp.bfloat16), b2t)[:n]
    return out.reshape(n * 8, cout)


# speedup vs baseline: 15.4982x; 15.4982x over previous
import functools

import numpy as np
import jax
import jax.numpy as jnp
from jax import lax
from jax.experimental import pallas as pl
from jax.experimental.pallas import tpu as pltpu
from jax.experimental.pallas import tpu_sc as plsc


_RES = 64
_BM = 2048
_NW = 32
_NT = 16
_CH = 256
_TSZ = _RES ** 3 + 1024
_SENT = _RES ** 3
_JUNK = _RES ** 3 + 2


def _child_parent_maps():
    A = np.zeros((2, 3, 3), dtype=np.float32)
    A[0, 0, 0] = 1.0
    A[0, 1, 1] = A[0, 1, 2] = 1.0
    A[1, 1, 0] = A[1, 1, 1] = 1.0
    A[1, 2, 2] = 1.0
    M = np.zeros((8, 27, 27), dtype=np.float32)
    for ox in range(2):
        for oy in range(2):
            for oz in range(2):
                o = 4 * ox + 2 * oy + oz
                for ex in range(3):
                    for ey in range(3):
                        for ez in range(3):
                            e = 9 * ex + 3 * ey + ez
                            for dx in range(3):
                                for dy in range(3):
                                    for dz in range(3):
                                        d = 9 * dx + 3 * dy + dz
                                        M[o, e, d] = (A[ox, ex, dx]
                                                      * A[oy, ey, dy]
                                                      * A[oz, ez, dz])
    return jnp.asarray(M)


def _as_i32_rows(x_bf16):
    u = lax.bitcast_convert_type(x_bf16, jnp.uint16)
    lo = u[:, 0::2].astype(jnp.int32)
    hi = u[:, 1::2].astype(jnp.int32)
    return (hi << 16) | lo


def _sc_conv1_gather(nkx, keys_pad, ids, feats_i32, ztab, npad):
    rtot = npad * 27
    nch = rtot // _CH
    steps = -(-nch // _NW)
    rf = feats_i32.shape[0]
    ts, fs, vs = _TSZ // _CH, rf // _CH, npad // _NT
    mesh = plsc.VectorSubcoreMesh(core_axis_name="c", subcore_axis_name="s")

    @functools.partial(
        pl.kernel, mesh=mesh,
        out_type=(jax.ShapeDtypeStruct((rtot,), jnp.int32),
                  jax.ShapeDtypeStruct((rtot // 8, 128), jnp.int32)),
        compiler_params=pltpu.CompilerParams(use_tc_tiling_on_sc=False),
        scratch_types=[
            pltpu.VMEM_SHARED((_TSZ,), jnp.int32),
            pltpu.VMEM_SHARED((rf, 16), jnp.int32),
            pltpu.VMEM((vs // 4,), jnp.int32),
            pltpu.VMEM((vs // 4,), jnp.int32),
            pltpu.VMEM((_CH,), jnp.int32),
            pltpu.VMEM((_CH,), jnp.int32),
            pltpu.VMEM((_CH, 16), jnp.int32),
            pltpu.VMEM((_CH // 8, 128), jnp.int32),
            pltpu.SemaphoreType.DMA,
        ],
    )
    def k(nkx_hbm, keys_hbm, ids_hbm, feats_hbm, ztab_hbm, g_hbm, gout_hbm,
          tab_sh, feats_sh, keys_v, ids_v, nk_v, g_v, rows_v, out128_v, sem):
        sid = lax.axis_index("s")
        wid = sid * 2 + lax.axis_index("c")

        def zstage(t, carry):
            c = sid + t * _NT

            @pl.when(c < ts)
            def _():
                pltpu.sync_copy(ztab_hbm.at[pl.ds(c * _CH, _CH)], g_v)
                pltpu.sync_copy(g_v, tab_sh.at[pl.ds(c * _CH, _CH)])

            return carry

        def fstage(t, carry):
            c = sid + t * _NT

            @pl.when(c < fs)
            def _():
                pltpu.sync_copy(feats_hbm.at[pl.ds(c * _CH, _CH)], rows_v)
                pltpu.sync_copy(rows_v, feats_sh.at[pl.ds(c * _CH, _CH)])

            return carry

        lax.fori_loop(0, -(-ts // _NT), zstage, 0)
        lax.fori_loop(0, -(-fs // _NT), fstage, 0)
        plsc.subcore_barrier()

        def scat(p, carry):
            off = sid * vs + p * (vs // 4)
            pltpu.sync_copy(keys_hbm.at[pl.ds(off, vs // 4)], keys_v)
            pltpu.sync_copy(ids_hbm.at[pl.ds(off, vs // 4)], ids_v)
            pltpu.async_copy(ids_v, tab_sh.at[keys_v], sem).wait()
            return carry

        lax.fori_loop(0, 4, scat, 0)
        plsc.subcore_barrier()

        def step(t, carry):
            c = wid + t * _NW

            @pl.when(c < nch)
            def _():
                pltpu.sync_copy(nkx_hbm.at[pl.ds(c * _CH, _CH)], nk_v)
                pltpu.async_copy(tab_sh.at[nk_v], g_v, sem).wait()
                pltpu.async_copy(feats_sh.at[g_v], rows_v, sem).wait()
                pltpu.sync_copy(g_v, g_hbm.at[pl.ds(c * _CH, _CH)])

                def repack(t, carry2):
                    for u in range(8):
                        out128_v[t, pl.ds(u * 16, 16)] = rows_v[t * 8 + u, :]
                    return carry2

                lax.fori_loop(0, _CH // 8, repack, 0)
                pltpu.sync_copy(
                    out128_v,
                    gout_hbm.at[pl.ds(c * (_CH // 8), _CH // 8)])

            return carry

        lax.fori_loop(0, steps, step, 0)

    return k(nkx, keys_pad, ids, feats_i32, ztab)


def _sc_conv2_gather(g, h1_i32, npad):
    rtot = npad * 27
    nch = rtot // _CH
    steps = -(-nch // _NW)
    rf = h1_i32.shape[0]
    fs = rf // _CH
    mesh = plsc.VectorSubcoreMesh(core_axis_name="c", subcore_axis_name="s")

    @functools.partial(
        pl.kernel, mesh=mesh,
        out_type=jax.ShapeDtypeStruct((rtot // 8, 128), jnp.int32),
        compiler_params=pltpu.CompilerParams(use_tc_tiling_on_sc=False),
        scratch_types=[
            pltpu.VMEM_SHARED((rf, 16), jnp.int32),
            pltpu.VMEM((_CH,), jnp.int32),
            pltpu.VMEM((_CH, 16), jnp.int32),
            pltpu.VMEM((_CH // 8, 128), jnp.int32),
            pltpu.SemaphoreType.DMA,
        ],
    )
    def k(g_hbm, h_hbm, gout_hbm, h_sh, g_v, rows_v, out128_v, sem):
        sid = lax.axis_index("s")
        wid = sid * 2 + lax.axis_index("c")

        def fstage(t, carry):
            c = sid + t * _NT

            @pl.when(c < fs)
            def _():
                pltpu.sync_copy(h_hbm.at[pl.ds(c * _CH, _CH)], rows_v)
                pltpu.sync_copy(rows_v, h_sh.at[pl.ds(c * _CH, _CH)])

            return carry

        lax.fori_loop(0, -(-fs // _NT), fstage, 0)
        plsc.subcore_barrier()

        def step(t, carry):
            c = wid + t * _NW

            @pl.when(c < nch)
            def _():
                pltpu.sync_copy(g_hbm.at[pl.ds(c * _CH, _CH)], g_v)
                pltpu.async_copy(h_sh.at[g_v], rows_v, sem).wait()

                def repack(t, carry2):
                    for u in range(8):
                        out128_v[t, pl.ds(u * 16, 16)] = rows_v[t * 8 + u, :]
                    return carry2

                lax.fori_loop(0, _CH // 8, repack, 0)
                pltpu.sync_copy(
                    out128_v,
                    gout_hbm.at[pl.ds(c * (_CH // 8), _CH // 8)])

            return carry

        lax.fori_loop(0, steps, step, 0)

    return k(g, h1_i32)


def _matmul_silu_kernel(x_ref, we_ref, wo_ref, b_ref, o_ref):
    x = x_ref[...]
    lo = lax.bitcast_convert_type(x << 16, jnp.float32).astype(jnp.bfloat16)
    hi = lax.bitcast_convert_type(x & jnp.int32(-65536),
                                  jnp.float32).astype(jnp.bfloat16)
    z = (jnp.dot(lo, we_ref[...], preferred_element_type=jnp.float32)
         + jnp.dot(hi, wo_ref[...], preferred_element_type=jnp.float32)
         + b_ref[...])
    o_ref[...] = z * (1.0 / (1.0 + jnp.exp(-z)))


def _matmul_silu(x_i32, w, b):
    m, kh = x_i32.shape
    co = w.shape[1]
    we = w[0::2].astype(jnp.bfloat16)
    wo = w[1::2].astype(jnp.bfloat16)
    return pl.pallas_call(
        _matmul_silu_kernel,
        grid=(m // _BM,),
        in_specs=[
            pl.BlockSpec((_BM, kh), lambda i: (i, 0)),
            pl.BlockSpec((kh, co), lambda i: (0, 0)),
            pl.BlockSpec((kh, co), lambda i: (0, 0)),
            pl.BlockSpec((1, co), lambda i: (0, 0)),
        ],
        out_specs=pl.BlockSpec((_BM, co), lambda i: (i, 0)),
        out_shape=jax.ShapeDtypeStruct((m, co), jnp.float32),
    )(x_i32, we, wo, b.reshape(1, co))


def kernel(feats, W1, b1, W2, b2, coords):
    n, cin = feats.shape
    cout = W2.shape[2]
    res = _RES
    npad = ((n + _BM - 1) // _BM) * _BM
    rf = npad + 512

    x, y, z = coords[:, 1], coords[:, 2], coords[:, 3]
    keys = (x * res + y) * res + z
    offs = np.array([[dx, dy, dz]
                     for dx in (-1, 0, 1)
                     for dy in (-1, 0, 1)
                     for dz in (-1, 0, 1)], dtype=np.int32)
    delta = jnp.asarray(offs[:, 0] * res * res + offs[:, 1] * res + offs[:, 2])
    q = coords[:, None, 1:4] + jnp.asarray(offs)[None, :, :]
    valid = jnp.all((q >= 0) & (q < res), axis=-1)
    nkx = jnp.where(valid, keys[:, None] + delta[None, :], _SENT)
    nkx = jnp.pad(nkx, ((0, npad - n), (0, 0)),
                  constant_values=_SENT).reshape(-1)

    keys_pad = jnp.pad(keys, (0, npad - n), constant_values=_JUNK)
    ids = jnp.arange(1, npad + 1, dtype=jnp.int32)
    ztab = jnp.zeros((_TSZ,), jnp.int32)

    fimg = jnp.concatenate(
        [jnp.zeros((1, cin), jnp.bfloat16),
         feats.astype(jnp.bfloat16),
         jnp.zeros((rf - 1 - n, cin), jnp.bfloat16)], 0)
    g, G1 = _sc_conv1_gather(nkx, keys_pad, ids, _as_i32_rows(fimg),
                             ztab, npad)
    G1 = G1.reshape(npad, 27 * cin // 2)
    W1r = W1.reshape(27 * cin, W1.shape[2])
    h1 = _matmul_silu(G1, W1r, b1)

    M = _child_parent_maps()
    W2eff = jnp.einsum('oed,dij->eioj', M, W2).reshape(27 * cin, 8 * cout)
    b2t = jnp.tile(b2, 8)
    himg = jnp.concatenate(
        [jnp.zeros((1, cin), jnp.bfloat16),
         h1.astype(jnp.bfloat16),
         jnp.zeros((rf - 1 - npad, cin), jnp.bfloat16)], 0)
    G2 = _sc_conv2_gather(g, _as_i32_rows(himg), npad)
    G2 = G2.reshape(npad, 27 * cin // 2)
    out = _matmul_silu(G2, W2eff, b2t)[:n]
    return out.reshape(n * 8, cout)
